# Initial kernel scaffold; baseline (speedup 1.0000x reference)
import functools

import numpy as np
import jax
import jax.numpy as jnp
from jax import lax
from jax.experimental import pallas as pl
from jax.experimental.pallas import tpu as pltpu
from jax.experimental.pallas import tpu_sc as plsc

_SPATIAL = [(80, 80), (40, 40), (20, 20)]
_B, _Q, _D, _H, _L, _P = 16, 300, 256, 8, 3, 4
_HD = _D // _H
_S = sum(h * w for h, w in _SPATIAL)
_BQ = _B * _Q
_NL = _H * _L * _P
_NC = 4 * _NL
_V = _B * _S * _H

_N_WORKERS = 32
_ROWS_PER_WORKER = _BQ // _N_WORKERS


def _build_consts():
    j = np.arange(_NL)
    h_idx = j // (_L * _P)
    l_idx = (j // _P) % _L
    wwl = np.array([w for _, w in _SPATIAL], np.float32)
    hhl = np.array([h for h, _ in _SPATIAL], np.float32)
    offl = np.cumsum([0] + [h * w for h, w in _SPATIAL])[:3].astype(np.float32)
    dx_c = [0.0, 1.0, 0.0, 1.0]
    dy_c = [0.0, 0.0, 1.0, 1.0]
    cm = np.zeros((8, _NC), np.float32)
    for c in range(4):
        sl = slice(c * _NL, (c + 1) * _NL)
        cm[0, sl] = wwl[l_idx]
        cm[1, sl] = hhl[l_idx]
        cm[2, sl] = offl[l_idx] * _H
        cm[3, sl] = h_idx
        cm[4, sl] = dx_c[c]
        cm[5, sl] = dy_c[c]
    t4 = np.zeros((_NL, _NC), np.float32)
    for c in range(4):
        t4[j, c * _NL + j] = 1.0
    segm = (j[:, None] // (_L * _P) == j[None, :] // (_L * _P)).astype(np.float32)
    return cm, t4, segm


_CM, _T4, _SEGM = _build_consts()


def _prep_body(hs_ref, rx_ref, ry_ref, wox_ref, box_ref, woy_ref, boy_ref,
               wat_ref, bat_ref, segm_ref, t4_ref, cm_ref, idx_ref, w_ref):
    f32 = jnp.float32
    hs = hs_ref[...]
    sox = jnp.dot(hs, wox_ref[...], preferred_element_type=f32) + box_ref[...]
    soy = jnp.dot(hs, woy_ref[...], preferred_element_type=f32) + boy_ref[...]
    logit = jnp.dot(hs, wat_ref[...], preferred_element_type=f32) + bat_ref[...]
    e = jnp.exp(logit)
    seg = jnp.dot(e, segm_ref[...], preferred_element_type=f32)
    aw = e / seg
    t4 = t4_ref[...]
    sx4 = jnp.dot(sox, t4, preferred_element_type=f32)
    sy4 = jnp.dot(soy, t4, preferred_element_type=f32)
    aw4 = jnp.dot(aw, t4, preferred_element_type=f32)
    cm = cm_ref[...]
    ww = cm[0:1, :]
    hh = cm[1:2, :]
    off8 = cm[2:3, :]
    hv = cm[3:4, :]
    dx = cm[4:5, :]
    dy = cm[5:6, :]
    rx = rx_ref[...]
    ry = ry_ref[...]
    gx = rx * ww + sx4 - 0.5
    gy = ry * hh + sy4 - 0.5
    x0 = jnp.floor(gx)
    y0 = jnp.floor(gy)
    fx = gx - x0
    fy = gy - y0
    xq = x0 + dx
    yq = y0 + dy
    wx = jnp.where(dx > 0.5, fx, 1.0 - fx)
    wy = jnp.where(dy > 0.5, fy, 1.0 - fy)
    valid = (xq >= 0.0) & (xq < ww) & (yq >= 0.0) & (yq < hh)
    wq = aw4 * wx * wy * jnp.where(valid, 1.0, 0.0)
    xc = jnp.clip(xq, 0.0, ww - 1.0)
    yc = jnp.clip(yq, 0.0, hh - 1.0)
    rblk = idx_ref.shape[0]
    rowi = lax.broadcasted_iota(jnp.int32, (rblk, _NC), 0) + pl.program_id(0) * rblk
    b = rowi // _Q
    idx_in_b = (off8 + (yc * ww + xc) * float(_H) + hv).astype(jnp.int32)
    idx_ref[...] = b * (_S * _H) + idx_in_b
    w_ref[...] = wq


def _mm_body(x_ref, w_ref, b_ref, o_ref):
    o_ref[...] = (jnp.dot(x_ref[...], w_ref[...],
                          preferred_element_type=jnp.float32) + b_ref[...])


def _matmul(x, wt, b, bm):
    m = x.shape[0]
    n = wt.shape[1]
    k = x.shape[1]
    grid = m // bm
    return pl.pallas_call(
        _mm_body,
        grid=(grid,),
        in_specs=[
            pl.BlockSpec((bm, k), lambda i: (i, 0)),
            pl.BlockSpec((k, n), lambda i: (0, 0)),
            pl.BlockSpec((1, n), lambda i: (0, 0)),
        ],
        out_specs=pl.BlockSpec((bm, n), lambda i: (i, 0)),
        out_shape=jax.ShapeDtypeStruct((m, n), jnp.float32),
    )(x, wt, b)


def _sc_gather_reduce(value_rows, idx4, wts):
    mesh = plsc.VectorSubcoreMesh(core_axis_name="c", subcore_axis_name="s")

    @functools.partial(
        pl.kernel,
        out_type=jax.ShapeDtypeStruct((_BQ, _D), jnp.float32),
        mesh=mesh,
        scratch_types=[
            pltpu.VMEM((4, _NL), jnp.int32),
            pltpu.VMEM((_NC, _HD), jnp.float32),
            pltpu.VMEM((_NC,), jnp.float32),
            pltpu.VMEM((_D,), jnp.float32),
            pltpu.SemaphoreType.DMA,
        ],
    )
    def k(value_hbm, idx_hbm, w_hbm, out_hbm, idx_v, g_v, w_v, out_v, sem):
        wid = lax.axis_index("s") * 2 + lax.axis_index("c")
        base = wid * _ROWS_PER_WORKER

        @pl.loop(0, _ROWS_PER_WORKER)
        def _(i):
            row = base + i
            pltpu.sync_copy(idx_hbm.at[row], idx_v)
            pltpu.sync_copy(w_hbm.at[row], w_v)
            copies = [
                pltpu.async_copy(value_hbm.at[idx_v.at[c]],
                                 g_v.at[pl.ds(c * _NL, _NL)], sem)
                for c in range(4)
            ]
            for cp in copies:
                cp.wait()
            for h in range(_H):
                a0 = jnp.zeros((16,), jnp.float32)
                a1 = jnp.zeros((16,), jnp.float32)
                for c in range(4):
                    for lp in range(_L * _P):
                        g = c * _NL + h * (_L * _P) + lp
                        ws = w_v[g]
                        a0 = a0 + ws * g_v[g, pl.ds(0, 16)]
                        a1 = a1 + ws * g_v[g, pl.ds(16, 16)]
                out_v[pl.ds(h * _HD, 16)] = a0
                out_v[pl.ds(h * _HD + 16, 16)] = a1
            pltpu.sync_copy(out_v, out_hbm.at[row])

    return k(value_rows, idx4, wts)


def kernel(hidden_states, encoder_hidden_states, reference_points,
           W_off, b_off, W_attn, b_attn, W_v, b_v, W_o, b_o):
    hs2 = hidden_states.reshape(_BQ, _D)
    enc2 = encoder_hidden_states.reshape(_B * _S, _D)

    refl = reference_points.reshape(_BQ, _L, 2)
    rx = jnp.broadcast_to(refl[:, None, None, :, None, 0],
                          (_BQ, 4, _H, _L, _P)).reshape(_BQ, _NC)
    ry = jnp.broadcast_to(refl[:, None, None, :, None, 1],
                          (_BQ, 4, _H, _L, _P)).reshape(_BQ, _NC)

    woff3 = W_off.reshape(_NL, 2, _D)
    boff2 = b_off.reshape(_NL, 2)
    wox = woff3[:, 0, :].T
    woy = woff3[:, 1, :].T
    box = boff2[:, 0].reshape(1, _NL)
    boy = boff2[:, 1].reshape(1, _NL)
    wat = W_attn.T
    bat = b_attn.reshape(1, _NL)

    cm = jnp.asarray(_CM)
    t4 = jnp.asarray(_T4)
    segm = jnp.asarray(_SEGM)

    rblk = 600
    idx, wts = pl.pallas_call(
        _prep_body,
        grid=(_BQ // rblk,),
        in_specs=[
            pl.BlockSpec((rblk, _D), lambda i: (i, 0)),
            pl.BlockSpec((rblk, _NC), lambda i: (i, 0)),
            pl.BlockSpec((rblk, _NC), lambda i: (i, 0)),
            pl.BlockSpec((_D, _NL), lambda i: (0, 0)),
            pl.BlockSpec((1, _NL), lambda i: (0, 0)),
            pl.BlockSpec((_D, _NL), lambda i: (0, 0)),
            pl.BlockSpec((1, _NL), lambda i: (0, 0)),
            pl.BlockSpec((_D, _NL), lambda i: (0, 0)),
            pl.BlockSpec((1, _NL), lambda i: (0, 0)),
            pl.BlockSpec((_NL, _NL), lambda i: (0, 0)),
            pl.BlockSpec((_NL, _NC), lambda i: (0, 0)),
            pl.BlockSpec((8, _NC), lambda i: (0, 0)),
        ],
        out_specs=[
            pl.BlockSpec((rblk, _NC), lambda i: (i, 0)),
            pl.BlockSpec((rblk, _NC), lambda i: (i, 0)),
        ],
        out_shape=(
            jax.ShapeDtypeStruct((_BQ, _NC), jnp.int32),
            jax.ShapeDtypeStruct((_BQ, _NC), jnp.float32),
        ),
    )(hs2, rx, ry, wox, box, woy, boy, wat, bat, segm, t4, cm)

    val = _matmul(enc2, W_v.T, b_v.reshape(1, _D), bm=4200)
    value_rows = val.reshape(_V, _HD)

    idx4 = idx.reshape(_BQ, 4, _NL)
    out2 = _sc_gather_reduce(value_rows, idx4, wts)

    fin = _matmul(out2, W_o.T, b_o.reshape(1, _D), bm=1200)
    return fin.reshape(_B, _Q, _D)


# speedup vs baseline: 15.4686x; 15.4686x over previous
import functools

import numpy as np
import jax
import jax.numpy as jnp
from jax import lax
from jax.experimental import pallas as pl
from jax.experimental.pallas import tpu as pltpu
from jax.experimental.pallas import tpu_sc as plsc

_SPATIAL = [(80, 80), (40, 40), (20, 20)]
_B, _Q, _D, _H, _L, _P = 16, 300, 256, 8, 3, 4
_HD = _D // _H
_S = sum(h * w for h, w in _SPATIAL)
_BQ = _B * _Q
_NL = _H * _L * _P
_NC = 4 * _NL
_V = _B * _S * _H

_N_WORKERS = 32
_ROWS_PER_WORKER = _BQ // _N_WORKERS


def _build_consts():
    j = np.arange(_NL)
    h_idx = j // (_L * _P)
    l_idx = (j // _P) % _L
    wwl = np.array([w for _, w in _SPATIAL], np.float32)
    hhl = np.array([h for h, _ in _SPATIAL], np.float32)
    offl = np.cumsum([0] + [h * w for h, w in _SPATIAL])[:3].astype(np.float32)
    dx_c = [0.0, 1.0, 0.0, 1.0]
    dy_c = [0.0, 0.0, 1.0, 1.0]
    cm = np.zeros((8, _NC), np.float32)
    for c in range(4):
        sl = slice(c * _NL, (c + 1) * _NL)
        cm[0, sl] = wwl[l_idx]
        cm[1, sl] = hhl[l_idx]
        cm[2, sl] = offl[l_idx] * _H
        cm[3, sl] = h_idx
        cm[4, sl] = dx_c[c]
        cm[5, sl] = dy_c[c]
    t4 = np.zeros((_NL, _NC), np.float32)
    for c in range(4):
        t4[j, c * _NL + j] = 1.0
    segm = (j[:, None] // (_L * _P) == j[None, :] // (_L * _P)).astype(np.float32)
    return cm, t4, segm


_CM, _T4, _SEGM = _build_consts()


def _prep_body(hs_ref, rx_ref, ry_ref, wox_ref, box_ref, woy_ref, boy_ref,
               wat_ref, bat_ref, segm_ref, t4_ref, cm_ref, idx_ref, w_ref):
    f32 = jnp.float32
    hs = hs_ref[...]
    sox = jnp.dot(hs, wox_ref[...], preferred_element_type=f32) + box_ref[...]
    soy = jnp.dot(hs, woy_ref[...], preferred_element_type=f32) + boy_ref[...]
    logit = jnp.dot(hs, wat_ref[...], preferred_element_type=f32) + bat_ref[...]
    e = jnp.exp(logit)
    seg = jnp.dot(e, segm_ref[...], preferred_element_type=f32)
    aw = e / seg
    t4 = t4_ref[...]
    sx4 = jnp.dot(sox, t4, preferred_element_type=f32)
    sy4 = jnp.dot(soy, t4, preferred_element_type=f32)
    aw4 = jnp.dot(aw, t4, preferred_element_type=f32)
    cm = cm_ref[...]
    ww = cm[0:1, :]
    hh = cm[1:2, :]
    off8 = cm[2:3, :]
    hv = cm[3:4, :]
    dx = cm[4:5, :]
    dy = cm[5:6, :]
    rx = rx_ref[...]
    ry = ry_ref[...]
    gx = rx * ww + sx4 - 0.5
    gy = ry * hh + sy4 - 0.5
    x0 = jnp.floor(gx)
    y0 = jnp.floor(gy)
    fx = gx - x0
    fy = gy - y0
    xq = x0 + dx
    yq = y0 + dy
    wx = jnp.where(dx > 0.5, fx, 1.0 - fx)
    wy = jnp.where(dy > 0.5, fy, 1.0 - fy)
    valid = (xq >= 0.0) & (xq < ww) & (yq >= 0.0) & (yq < hh)
    wq = aw4 * wx * wy * jnp.where(valid, 1.0, 0.0)
    xc = jnp.clip(xq, 0.0, ww - 1.0)
    yc = jnp.clip(yq, 0.0, hh - 1.0)
    rblk = idx_ref.shape[0]
    rowi = lax.broadcasted_iota(jnp.int32, (rblk, _NC), 0) + pl.program_id(0) * rblk
    b = rowi // _Q
    idx_in_b = (off8 + (yc * ww + xc) * float(_H) + hv).astype(jnp.int32)
    idx_ref[...] = b * (_S * _H) + idx_in_b
    w_ref[...] = wq


def _mm_body(x_ref, w_ref, b_ref, o_ref):
    o_ref[...] = (jnp.dot(x_ref[...], w_ref[...],
                          preferred_element_type=jnp.float32) + b_ref[...])


def _matmul(x, wt, b, bm):
    m = x.shape[0]
    n = wt.shape[1]
    k = x.shape[1]
    grid = m // bm
    return pl.pallas_call(
        _mm_body,
        grid=(grid,),
        in_specs=[
            pl.BlockSpec((bm, k), lambda i: (i, 0)),
            pl.BlockSpec((k, n), lambda i: (0, 0)),
            pl.BlockSpec((1, n), lambda i: (0, 0)),
        ],
        out_specs=pl.BlockSpec((bm, n), lambda i: (i, 0)),
        out_shape=jax.ShapeDtypeStruct((m, n), jnp.float32),
    )(x, wt, b)


def _sc_gather_reduce(value_rows, idx4, wts):
    mesh = plsc.VectorSubcoreMesh(core_axis_name="c", subcore_axis_name="s")

    @functools.partial(
        pl.kernel,
        out_type=jax.ShapeDtypeStruct((_BQ, _D), jnp.float32),
        mesh=mesh,
        scratch_types=[
            pltpu.VMEM((4, _NL), jnp.int32),
            pltpu.VMEM((_NC, _HD), jnp.float32),
            pltpu.VMEM((_NC,), jnp.float32),
            pltpu.VMEM((_D,), jnp.float32),
            pltpu.SemaphoreType.DMA,
        ],
        compiler_params=pltpu.CompilerParams(use_tc_tiling_on_sc=False),
    )
    def k(value_hbm, idx_hbm, w_hbm, out_hbm, idx_v, g_v, w_v, out_v, sem):
        wid = lax.axis_index("s") * 2 + lax.axis_index("c")
        base = wid * _ROWS_PER_WORKER

        @pl.loop(0, _ROWS_PER_WORKER)
        def _(i):
            row = base + i
            pltpu.sync_copy(idx_hbm.at[row], idx_v)
            pltpu.sync_copy(w_hbm.at[row], w_v)
            copies = [
                pltpu.async_copy(value_hbm.at[idx_v.at[c]],
                                 g_v.at[pl.ds(c * _NL, _NL)], sem)
                for c in range(4)
            ]
            for cp in copies:
                cp.wait()
            wchunks = [w_v[pl.ds(kk * 16, 16)] for kk in range(_NC // 16)]
            for h in range(_H):
                a0 = jnp.zeros((16,), jnp.float32)
                a1 = jnp.zeros((16,), jnp.float32)
                for c in range(4):
                    for lp in range(_L * _P):
                        g = c * _NL + h * (_L * _P) + lp
                        ws = wchunks[g // 16][g % 16]
                        a0 = a0 + ws * g_v[g, pl.ds(0, 16)]
                        a1 = a1 + ws * g_v[g, pl.ds(16, 16)]
                out_v[pl.ds(h * _HD, 16)] = a0
                out_v[pl.ds(h * _HD + 16, 16)] = a1
            pltpu.sync_copy(out_v, out_hbm.at[row])

    return k(value_rows, idx4, wts)


def kernel(hidden_states, encoder_hidden_states, reference_points,
           W_off, b_off, W_attn, b_attn, W_v, b_v, W_o, b_o):
    hs2 = hidden_states.reshape(_BQ, _D)
    enc2 = encoder_hidden_states.reshape(_B * _S, _D)

    refl = reference_points.reshape(_BQ, _L, 2)
    rx = jnp.broadcast_to(refl[:, None, None, :, None, 0],
                          (_BQ, 4, _H, _L, _P)).reshape(_BQ, _NC)
    ry = jnp.broadcast_to(refl[:, None, None, :, None, 1],
                          (_BQ, 4, _H, _L, _P)).reshape(_BQ, _NC)

    woff3 = W_off.reshape(_NL, 2, _D)
    boff2 = b_off.reshape(_NL, 2)
    wox = woff3[:, 0, :].T
    woy = woff3[:, 1, :].T
    box = boff2[:, 0].reshape(1, _NL)
    boy = boff2[:, 1].reshape(1, _NL)
    wat = W_attn.T
    bat = b_attn.reshape(1, _NL)

    cm = jnp.asarray(_CM)
    t4 = jnp.asarray(_T4)
    segm = jnp.asarray(_SEGM)

    rblk = 600
    idx, wts = pl.pallas_call(
        _prep_body,
        grid=(_BQ // rblk,),
        in_specs=[
            pl.BlockSpec((rblk, _D), lambda i: (i, 0)),
            pl.BlockSpec((rblk, _NC), lambda i: (i, 0)),
            pl.BlockSpec((rblk, _NC), lambda i: (i, 0)),
            pl.BlockSpec((_D, _NL), lambda i: (0, 0)),
            pl.BlockSpec((1, _NL), lambda i: (0, 0)),
            pl.BlockSpec((_D, _NL), lambda i: (0, 0)),
            pl.BlockSpec((1, _NL), lambda i: (0, 0)),
            pl.BlockSpec((_D, _NL), lambda i: (0, 0)),
            pl.BlockSpec((1, _NL), lambda i: (0, 0)),
            pl.BlockSpec((_NL, _NL), lambda i: (0, 0)),
            pl.BlockSpec((_NL, _NC), lambda i: (0, 0)),
            pl.BlockSpec((8, _NC), lambda i: (0, 0)),
        ],
        out_specs=[
            pl.BlockSpec((rblk, _NC), lambda i: (i, 0)),
            pl.BlockSpec((rblk, _NC), lambda i: (i, 0)),
        ],
        out_shape=(
            jax.ShapeDtypeStruct((_BQ, _NC), jnp.int32),
            jax.ShapeDtypeStruct((_BQ, _NC), jnp.float32),
        ),
    )(hs2, rx, ry, wox, box, woy, boy, wat, bat, segm, t4, cm)

    val = _matmul(enc2, W_v.T, b_v.reshape(1, _D), bm=4200)
    value_rows = val.reshape(_V, _HD)

    idx4 = idx.reshape(_BQ, 4, _NL)
    out2 = _sc_gather_reduce(value_rows, idx4, wts)

    fin = _matmul(out2, W_o.T, b_o.reshape(1, _D), bm=1200)
    return fin.reshape(_B, _Q, _D)


# speedup vs baseline: 16.9215x; 1.0939x over previous
import functools

import numpy as np
import jax
import jax.numpy as jnp
from jax import lax
from jax.experimental import pallas as pl
from jax.experimental.pallas import tpu as pltpu
from jax.experimental.pallas import tpu_sc as plsc

_SPATIAL = [(80, 80), (40, 40), (20, 20)]
_B, _Q, _D, _H, _L, _P = 16, 300, 256, 8, 3, 4
_HD = _D // _H
_S = sum(h * w for h, w in _SPATIAL)
_BQ = _B * _Q
_NL = _H * _L * _P
_NC = 4 * _NL
_V = _B * _S * _H

_N_WORKERS = 32
_ROWS_PER_WORKER = _BQ // _N_WORKERS


def _build_consts():
    j = np.arange(_NL)
    h_idx = j // (_L * _P)
    l_idx = (j // _P) % _L
    wwl = np.array([w for _, w in _SPATIAL], np.float32)
    hhl = np.array([h for h, _ in _SPATIAL], np.float32)
    offl = np.cumsum([0] + [h * w for h, w in _SPATIAL])[:3].astype(np.float32)
    dx_c = [0.0, 1.0, 0.0, 1.0]
    dy_c = [0.0, 0.0, 1.0, 1.0]
    cm = np.zeros((8, _NC), np.float32)
    for c in range(4):
        sl = slice(c * _NL, (c + 1) * _NL)
        cm[0, sl] = wwl[l_idx]
        cm[1, sl] = hhl[l_idx]
        cm[2, sl] = offl[l_idx] * _H
        cm[3, sl] = h_idx
        cm[4, sl] = dx_c[c]
        cm[5, sl] = dy_c[c]
    t4 = np.zeros((_NL, _NC), np.float32)
    for c in range(4):
        t4[j, c * _NL + j] = 1.0
    segm = (j[:, None] // (_L * _P) == j[None, :] // (_L * _P)).astype(np.float32)
    return cm, t4, segm


_CM, _T4, _SEGM = _build_consts()


def _prep_body(hs_ref, rx_ref, ry_ref, wox_ref, box_ref, woy_ref, boy_ref,
               wat_ref, bat_ref, segm_ref, t4_ref, cm_ref, idx_ref, w_ref):
    f32 = jnp.float32
    hs = hs_ref[...]
    sox = jnp.dot(hs, wox_ref[...], preferred_element_type=f32) + box_ref[...]
    soy = jnp.dot(hs, woy_ref[...], preferred_element_type=f32) + boy_ref[...]
    logit = jnp.dot(hs, wat_ref[...], preferred_element_type=f32) + bat_ref[...]
    e = jnp.exp(logit)
    seg = jnp.dot(e, segm_ref[...], preferred_element_type=f32)
    aw = e / seg
    t4 = t4_ref[...]
    sx4 = jnp.dot(sox, t4, preferred_element_type=f32)
    sy4 = jnp.dot(soy, t4, preferred_element_type=f32)
    aw4 = jnp.dot(aw, t4, preferred_element_type=f32)
    cm = cm_ref[...]
    ww = cm[0:1, :]
    hh = cm[1:2, :]
    off8 = cm[2:3, :]
    hv = cm[3:4, :]
    dx = cm[4:5, :]
    dy = cm[5:6, :]
    rx = rx_ref[...]
    ry = ry_ref[...]
    gx = rx * ww + sx4 - 0.5
    gy = ry * hh + sy4 - 0.5
    x0 = jnp.floor(gx)
    y0 = jnp.floor(gy)
    fx = gx - x0
    fy = gy - y0
    xq = x0 + dx
    yq = y0 + dy
    wx = jnp.where(dx > 0.5, fx, 1.0 - fx)
    wy = jnp.where(dy > 0.5, fy, 1.0 - fy)
    valid = (xq >= 0.0) & (xq < ww) & (yq >= 0.0) & (yq < hh)
    wq = aw4 * wx * wy * jnp.where(valid, 1.0, 0.0)
    xc = jnp.clip(xq, 0.0, ww - 1.0)
    yc = jnp.clip(yq, 0.0, hh - 1.0)
    rblk = idx_ref.shape[0]
    rowi = lax.broadcasted_iota(jnp.int32, (rblk, _NC), 0) + pl.program_id(0) * rblk
    b = rowi // _Q
    idx_in_b = (off8 + (yc * ww + xc) * float(_H) + hv).astype(jnp.int32)
    idx_ref[...] = b * (_S * _H) + idx_in_b
    w_ref[...] = wq


def _mm_body(x_ref, w_ref, b_ref, o_ref):
    o_ref[...] = (jnp.dot(x_ref[...], w_ref[...],
                          preferred_element_type=jnp.float32) + b_ref[...])


def _matmul(x, wt, b, bm):
    m = x.shape[0]
    n = wt.shape[1]
    k = x.shape[1]
    grid = m // bm
    return pl.pallas_call(
        _mm_body,
        grid=(grid,),
        in_specs=[
            pl.BlockSpec((bm, k), lambda i: (i, 0)),
            pl.BlockSpec((k, n), lambda i: (0, 0)),
            pl.BlockSpec((1, n), lambda i: (0, 0)),
        ],
        out_specs=pl.BlockSpec((bm, n), lambda i: (i, 0)),
        out_shape=jax.ShapeDtypeStruct((m, n), jnp.float32),
    )(x, wt, b)


_KB = 3
_NT = _ROWS_PER_WORKER // _KB


def _sc_gather_reduce(value_rows, idx4, wts):
    mesh = plsc.VectorSubcoreMesh(core_axis_name="c", subcore_axis_name="s")

    @functools.partial(
        pl.kernel,
        out_type=jax.ShapeDtypeStruct((_BQ, _D), jnp.float32),
        mesh=mesh,
        scratch_types=[
            pltpu.VMEM((2, _KB, 4, _NL), jnp.int32),
            pltpu.VMEM((2, _KB, _NC, _HD), jnp.float32),
            pltpu.VMEM((2, _KB, _NC), jnp.float32),
            pltpu.VMEM((2, _KB, _D), jnp.float32),
            (pltpu.SemaphoreType.DMA, pltpu.SemaphoreType.DMA),
            (pltpu.SemaphoreType.DMA, pltpu.SemaphoreType.DMA),
            (pltpu.SemaphoreType.DMA, pltpu.SemaphoreType.DMA),
        ],
        compiler_params=pltpu.CompilerParams(use_tc_tiling_on_sc=False),
    )
    def k(value_hbm, idx_hbm, w_hbm, out_hbm, idx_v, g_v, w_v, out_v,
          sem_m, sem_g, sem_o):
        wid = lax.axis_index("s") * 2 + lax.axis_index("c")
        base = wid * _ROWS_PER_WORKER
        last0 = _BQ - _KB

        def row0_of(t):
            return jnp.minimum(base + t * _KB, last0)

        def start_meta(t, buf):
            r0 = row0_of(t)
            pltpu.async_copy(idx_hbm.at[pl.ds(r0, _KB)], idx_v.at[buf],
                             sem_m[buf])
            pltpu.async_copy(w_hbm.at[pl.ds(r0, _KB)], w_v.at[buf],
                             sem_m[buf])

        def wait_meta(buf):
            pltpu.make_async_copy(idx_hbm.at[pl.ds(0, _KB)],
                                  idx_v.at[buf], sem_m[buf]).wait()
            pltpu.make_async_copy(w_hbm.at[pl.ds(0, _KB)],
                                  w_v.at[buf], sem_m[buf]).wait()

        def start_gather(buf):
            for r in range(_KB):
                for c in range(4):
                    pltpu.async_copy(
                        value_hbm.at[idx_v.at[buf, r, c]],
                        g_v.at[buf, r, pl.ds(c * _NL, _NL)], sem_g[buf])

        def wait_gather(buf):
            for r in range(_KB):
                for c in range(4):
                    pltpu.make_async_copy(
                        value_hbm.at[pl.ds(0, _NL)],
                        g_v.at[buf, r, pl.ds(c * _NL, _NL)],
                        sem_g[buf]).wait()

        def start_out(t, buf):
            pltpu.async_copy(out_v.at[buf],
                             out_hbm.at[pl.ds(row0_of(t), _KB)], sem_o[buf])

        def wait_out(buf):
            pltpu.make_async_copy(out_v.at[buf],
                                  out_hbm.at[pl.ds(0, _KB)], sem_o[buf]).wait()

        def compute(buf):
            for r in range(_KB):
                wchunks = [w_v[buf, r, pl.ds(kk * 16, 16)]
                           for kk in range(_NC // 16)]
                for h in range(_H):
                    a0 = jnp.zeros((16,), jnp.float32)
                    a1 = jnp.zeros((16,), jnp.float32)
                    for c in range(4):
                        for lp in range(_L * _P):
                            g = c * _NL + h * (_L * _P) + lp
                            ws = wchunks[g // 16][g % 16]
                            a0 = a0 + ws * g_v[buf, r, g, pl.ds(0, 16)]
                            a1 = a1 + ws * g_v[buf, r, g, pl.ds(16, 16)]
                    out_v[buf, r, pl.ds(h * _HD, 16)] = a0
                    out_v[buf, r, pl.ds(h * _HD + 16, 16)] = a1

        start_meta(0, 0)
        wait_meta(0)
        start_gather(0)
        start_meta(1, 1)

        @pl.loop(0, _NT // 2)
        def _(u):
            t0 = u * 2
            for buf in (0, 1):
                t = t0 + buf
                nb = 1 - buf
                wait_meta(nb)
                start_gather(nb)
                wait_gather(buf)
                @pl.when(t >= 2)
                def _():
                    wait_out(buf)
                compute(buf)
                start_out(t, buf)
                start_meta(t + 2, buf)

        wait_meta(1)
        wait_gather(0)
        wait_out(0)
        wait_out(1)

    return k(value_rows, idx4, wts)


def kernel(hidden_states, encoder_hidden_states, reference_points,
           W_off, b_off, W_attn, b_attn, W_v, b_v, W_o, b_o):
    hs2 = hidden_states.reshape(_BQ, _D)
    enc2 = encoder_hidden_states.reshape(_B * _S, _D)

    refl = reference_points.reshape(_BQ, _L, 2)
    rx = jnp.broadcast_to(refl[:, None, None, :, None, 0],
                          (_BQ, 4, _H, _L, _P)).reshape(_BQ, _NC)
    ry = jnp.broadcast_to(refl[:, None, None, :, None, 1],
                          (_BQ, 4, _H, _L, _P)).reshape(_BQ, _NC)

    woff3 = W_off.reshape(_NL, 2, _D)
    boff2 = b_off.reshape(_NL, 2)
    wox = woff3[:, 0, :].T
    woy = woff3[:, 1, :].T
    box = boff2[:, 0].reshape(1, _NL)
    boy = boff2[:, 1].reshape(1, _NL)
    wat = W_attn.T
    bat = b_attn.reshape(1, _NL)

    cm = jnp.asarray(_CM)
    t4 = jnp.asarray(_T4)
    segm = jnp.asarray(_SEGM)

    rblk = 600
    idx, wts = pl.pallas_call(
        _prep_body,
        grid=(_BQ // rblk,),
        in_specs=[
            pl.BlockSpec((rblk, _D), lambda i: (i, 0)),
            pl.BlockSpec((rblk, _NC), lambda i: (i, 0)),
            pl.BlockSpec((rblk, _NC), lambda i: (i, 0)),
            pl.BlockSpec((_D, _NL), lambda i: (0, 0)),
            pl.BlockSpec((1, _NL), lambda i: (0, 0)),
            pl.BlockSpec((_D, _NL), lambda i: (0, 0)),
            pl.BlockSpec((1, _NL), lambda i: (0, 0)),
            pl.BlockSpec((_D, _NL), lambda i: (0, 0)),
            pl.BlockSpec((1, _NL), lambda i: (0, 0)),
            pl.BlockSpec((_NL, _NL), lambda i: (0, 0)),
            pl.BlockSpec((_NL, _NC), lambda i: (0, 0)),
            pl.BlockSpec((8, _NC), lambda i: (0, 0)),
        ],
        out_specs=[
            pl.BlockSpec((rblk, _NC), lambda i: (i, 0)),
            pl.BlockSpec((rblk, _NC), lambda i: (i, 0)),
        ],
        out_shape=(
            jax.ShapeDtypeStruct((_BQ, _NC), jnp.int32),
            jax.ShapeDtypeStruct((_BQ, _NC), jnp.float32),
        ),
    )(hs2, rx, ry, wox, box, woy, boy, wat, bat, segm, t4, cm)

    val = _matmul(enc2, W_v.T, b_v.reshape(1, _D), bm=4200)
    value_rows = val.reshape(_V, _HD)

    idx4 = idx.reshape(_BQ, 4, _NL)
    out2 = _sc_gather_reduce(value_rows, idx4, wts)

    fin = _matmul(out2, W_o.T, b_o.reshape(1, _D), bm=1200)
    return fin.reshape(_B, _Q, _D)
